# baseline (speedup 1.0000x reference)
import functools

import jax
import jax.numpy as jnp
from jax import lax
from jax.experimental import pallas as pl
from jax.experimental.pallas import tpu as pltpu


def _double_conv_kernel(x_ref, w1_ref, b1_ref, w2_ref, b2_ref, out_ref,
                        xs_ref, col_ref, wp1_ref, wp2_ref, ml_ref, mr_ref,
                        *, k, padding, W, L, SL, cin, cout):
    K = k * k * cin
    kk = k * k
    pid = pl.program_id(0)

    @pl.when((pid == 0) | (pid == 1) | (pid == pl.num_programs(0) // 2))
    def _init():
        j = lax.broadcasted_iota(jnp.int32, (K, K), 0)
        m = lax.broadcasted_iota(jnp.int32, (K, K), 1)
        perm = ((j % kk) * cin + (j // kk)) == m
        p = jnp.where(perm, jnp.float32(1), jnp.float32(0)).astype(jnp.bfloat16)
        wp1_ref[...] = jnp.dot(w1_ref[...].astype(jnp.bfloat16), p,
                               preferred_element_type=jnp.float32
                               ).astype(jnp.bfloat16)
        wp2_ref[...] = jnp.dot(w2_ref[...].astype(jnp.bfloat16), p,
                               preferred_element_type=jnp.float32
                               ).astype(jnp.bfloat16)
        lane = lax.broadcasted_iota(jnp.int32, (8, L), 1) % W
        ml_ref[...] = jnp.where(lane >= 1, jnp.float32(1),
                                jnp.float32(0)).astype(jnp.bfloat16)
        mr_ref[...] = jnp.where(lane <= W - 2, jnp.float32(1),
                                jnp.float32(0)).astype(jnp.bfloat16)

    if SL > 0:
        zeros = jnp.zeros((cin, SL), jnp.bfloat16)
        xs_ref[:, 0:SL] = zeros
        xs_ref[:, SL + L:SL + L + SL] = zeros
    xs_ref[:, SL:SL + L] = x_ref[0].astype(jnp.bfloat16)

    mask_rows = {0: ml_ref[0:1, :], k - 1: mr_ref[0:1, :]}

    def build_col():
        for kh in range(k):
            for kw in range(k):
                tap = kh * k + kw
                start = SL + (kh - padding) * W + (kw - padding)
                patch = xs_ref[:, start:start + L]
                if kw != padding:
                    patch = patch * mask_rows[kw]
                col_ref[tap * cin:(tap + 1) * cin, :] = patch

    build_col()
    h1 = jnp.dot(wp1_ref[...], col_ref[...],
                 preferred_element_type=jnp.float32) + b1_ref[...]
    h1 = jnp.where(h1 > 0, h1, 0.01 * h1)

    xs_ref[:, SL:SL + L] = h1.astype(jnp.bfloat16)
    build_col()
    out = jnp.dot(wp2_ref[...], col_ref[...],
                  preferred_element_type=jnp.float32) + b2_ref[...]
    out_ref[0] = out


def kernel(x, w1, b1, w2, b2):
    B, cin, H, W = x.shape
    cout = w2.shape[0]
    k = w1.shape[2]
    padding = (k - 1) // 2
    L = H * W
    K = k * k * cin
    S = padding * W + padding
    SL = ((S + 127) // 128) * 128 if S > 0 else 0

    x_flat = x.reshape(B, cin, L)
    w1f = w1.reshape(cin, K)
    w2f = w2.reshape(cout, K)
    b1v = b1.reshape(cin, 1)
    b2v = b2.reshape(cout, 1)

    _kernel_fn = functools.partial(
        _double_conv_kernel, k=k, padding=padding, W=W, L=L, SL=SL,
        cin=cin, cout=cout)

    flops = 2 * B * K * L * (cin + cout)
    bytes_accessed = 4 * (B * cin * L + B * cout * L + (cin + cout) * (K + 1))

    out_flat = pl.pallas_call(
        _kernel_fn,
        out_shape=jax.ShapeDtypeStruct((B, cout, L), jnp.float32),
        grid=(B,),
        in_specs=[
            pl.BlockSpec((1, cin, L), lambda b: (b, 0, 0)),
            pl.BlockSpec((cin, K), lambda b: (0, 0)),
            pl.BlockSpec((cin, 1), lambda b: (0, 0)),
            pl.BlockSpec((cout, K), lambda b: (0, 0)),
            pl.BlockSpec((cout, 1), lambda b: (0, 0)),
        ],
        out_specs=pl.BlockSpec((1, cout, L), lambda b: (b, 0, 0)),
        scratch_shapes=[
            pltpu.VMEM((cin, SL + L + SL), jnp.bfloat16),
            pltpu.VMEM((K, L), jnp.bfloat16),
            pltpu.VMEM((cin, K), jnp.bfloat16),
            pltpu.VMEM((cout, K), jnp.bfloat16),
            pltpu.VMEM((8, L), jnp.bfloat16),
            pltpu.VMEM((8, L), jnp.bfloat16),
        ],
        compiler_params=pltpu.CompilerParams(
            dimension_semantics=("parallel",)),
        cost_estimate=pl.CostEstimate(
            flops=flops, transcendentals=0, bytes_accessed=bytes_accessed),
    )(x_flat, w1f, b1v, w2f, b2v)

    return out_flat.reshape(B, cout, H, W)


# speedup vs baseline: 1.0480x; 1.0480x over previous
import functools

import jax
import jax.numpy as jnp
from jax import lax
from jax.experimental import pallas as pl
from jax.experimental.pallas import tpu as pltpu


def _double_conv_kernel(x_ref, w1_ref, b1_ref, w2_ref, b2_ref, out_ref,
                        xs_ref, col_ref, wp1_ref, wp2_ref, ml_ref, mr_ref,
                        *, k, padding, W, L, SL, cin, cout, NB):
    K = k * k * cin
    kk = k * k
    KH = k * cin
    P = SL + L
    pid = pl.program_id(0)

    @pl.when((pid == 0) | (pid == 1) | (pid == pl.num_programs(0) // 2))
    def _init():
        j = lax.broadcasted_iota(jnp.int32, (K, K), 0)
        m = lax.broadcasted_iota(jnp.int32, (K, K), 1)
        perm = ((j % kk) * cin + (j // kk)) == m
        p = jnp.where(perm, jnp.float32(1), jnp.float32(0)).astype(jnp.bfloat16)
        wp1_ref[...] = jnp.dot(w1_ref[...].astype(jnp.bfloat16), p,
                               preferred_element_type=jnp.float32
                               ).astype(jnp.bfloat16)
        wp2_ref[...] = jnp.dot(w2_ref[...].astype(jnp.bfloat16), p,
                               preferred_element_type=jnp.float32
                               ).astype(jnp.bfloat16)
        lane = lax.broadcasted_iota(jnp.int32, (8, L), 1) % W
        ml_ref[...] = jnp.where(lane >= 1, jnp.float32(1),
                                jnp.float32(0)).astype(jnp.bfloat16)
        mr_ref[...] = jnp.where(lane <= W - 2, jnp.float32(1),
                                jnp.float32(0)).astype(jnp.bfloat16)

    xs_ref[:, 0:SL] = jnp.zeros((cin, SL), jnp.bfloat16)
    for b in range(NB):
        base = SL + b * P
        xs_ref[:, base:base + L] = x_ref[b].astype(jnp.bfloat16)
        xs_ref[:, base + L:base + L + SL] = jnp.zeros((cin, SL), jnp.bfloat16)

    mask_rows = {0: ml_ref[0:1, :], k - 1: mr_ref[0:1, :]}

    def build_kh(kh):
        for kw in range(k):
            tap = kh * k + kw
            shift = (kh - padding) * W + (kw - padding)
            for b in range(NB):
                patch = xs_ref[:, SL + b * P + shift:SL + b * P + shift + L]
                if kw != padding:
                    patch = patch * mask_rows[kw]
                col_ref[tap * cin:(tap + 1) * cin, b * L:(b + 1) * L] = patch

    def conv(w_ref, b_ref):
        acc = None
        for kh in range(k):
            build_kh(kh)
            d = jnp.dot(w_ref[:, kh * KH:(kh + 1) * KH],
                        col_ref[kh * KH:(kh + 1) * KH, :],
                        preferred_element_type=jnp.float32)
            acc = d if acc is None else acc + d
        return acc + b_ref[...]

    h1 = conv(wp1_ref, b1_ref)
    h1 = jnp.where(h1 > 0, h1, 0.01 * h1)
    h1 = h1.astype(jnp.bfloat16)
    for b in range(NB):
        base = SL + b * P
        xs_ref[:, base:base + L] = h1[:, b * L:(b + 1) * L]

    out = conv(wp2_ref, b2_ref)
    for b in range(NB):
        out_ref[b] = out[:, b * L:(b + 1) * L]


def kernel(x, w1, b1, w2, b2):
    B, cin, H, W = x.shape
    cout = w2.shape[0]
    k = w1.shape[2]
    padding = (k - 1) // 2
    L = H * W
    K = k * k * cin
    S = padding * W + padding
    SL = ((S + 127) // 128) * 128 if S > 0 else 0
    NB = 2 if B % 2 == 0 else 1

    x_flat = x.reshape(B, cin, L)
    w1f = w1.reshape(cin, K)
    w2f = w2.reshape(cout, K)
    b1v = b1.reshape(cin, 1)
    b2v = b2.reshape(cout, 1)

    _kernel_fn = functools.partial(
        _double_conv_kernel, k=k, padding=padding, W=W, L=L, SL=SL,
        cin=cin, cout=cout, NB=NB)

    flops = 2 * B * K * L * (cin + cout)
    bytes_accessed = 4 * (B * cin * L + B * cout * L + (cin + cout) * (K + 1))

    out_flat = pl.pallas_call(
        _kernel_fn,
        out_shape=jax.ShapeDtypeStruct((B, cout, L), jnp.float32),
        grid=(B // NB,),
        in_specs=[
            pl.BlockSpec((NB, cin, L), lambda b: (b, 0, 0)),
            pl.BlockSpec((cin, K), lambda b: (0, 0)),
            pl.BlockSpec((cin, 1), lambda b: (0, 0)),
            pl.BlockSpec((cout, K), lambda b: (0, 0)),
            pl.BlockSpec((cout, 1), lambda b: (0, 0)),
        ],
        out_specs=pl.BlockSpec((NB, cout, L), lambda b: (b, 0, 0)),
        scratch_shapes=[
            pltpu.VMEM((cin, SL + NB * (L + SL)), jnp.bfloat16),
            pltpu.VMEM((K, NB * L), jnp.bfloat16),
            pltpu.VMEM((cin, K), jnp.bfloat16),
            pltpu.VMEM((cout, K), jnp.bfloat16),
            pltpu.VMEM((8, L), jnp.bfloat16),
            pltpu.VMEM((8, L), jnp.bfloat16),
        ],
        compiler_params=pltpu.CompilerParams(
            dimension_semantics=("parallel",)),
        cost_estimate=pl.CostEstimate(
            flops=flops, transcendentals=0, bytes_accessed=bytes_accessed),
    )(x_flat, w1f, b1v, w2f, b2v)

    return out_flat.reshape(B, cout, H, W)


# speedup vs baseline: 1.0854x; 1.0357x over previous
import functools

import jax
import jax.numpy as jnp
from jax import lax
from jax.experimental import pallas as pl
from jax.experimental.pallas import tpu as pltpu


def _double_conv_kernel(x_ref, w1_ref, b1_ref, w2_ref, b2_ref, out_ref,
                        xs_ref, wp1_ref, wp2_ref, ml_ref, mr_ref,
                        *, k, padding, W, L, SL, cin, cout, NB):
    K = k * k * cin
    kk = k * k
    KH = k * cin
    P = SL + L
    pid = pl.program_id(0)

    @pl.when((pid == 0) | (pid == 1) | (pid == pl.num_programs(0) // 2))
    def _init():
        j = lax.broadcasted_iota(jnp.int32, (K, K), 0)
        m = lax.broadcasted_iota(jnp.int32, (K, K), 1)
        perm = ((j % kk) * cin + (j // kk)) == m
        p = jnp.where(perm, jnp.float32(1), jnp.float32(0)).astype(jnp.bfloat16)
        wp1_ref[...] = jnp.dot(w1_ref[...].astype(jnp.bfloat16), p,
                               preferred_element_type=jnp.float32
                               ).astype(jnp.bfloat16)
        wp2_ref[...] = jnp.dot(w2_ref[...].astype(jnp.bfloat16), p,
                               preferred_element_type=jnp.float32
                               ).astype(jnp.bfloat16)
        lane = lax.broadcasted_iota(jnp.int32, (8, L), 1) % W
        ml_ref[...] = jnp.where(lane >= 1, jnp.float32(1),
                                jnp.float32(0)).astype(jnp.bfloat16)
        mr_ref[...] = jnp.where(lane <= W - 2, jnp.float32(1),
                                jnp.float32(0)).astype(jnp.bfloat16)

    xs_ref[:, 0:SL] = jnp.zeros((cin, SL), jnp.bfloat16)
    for b in range(NB):
        base = SL + b * P
        xs_ref[:, base:base + L] = x_ref[b].astype(jnp.bfloat16)
        xs_ref[:, base + L:base + L + SL] = jnp.zeros((cin, SL), jnp.bfloat16)

    mask_rows = {0: ml_ref[0:1, :], k - 1: mr_ref[0:1, :]}

    def patch_kh(kh):
        rows = []
        for kw in range(k):
            shift = (kh - padding) * W + (kw - padding)
            parts = []
            for b in range(NB):
                patch = xs_ref[:, SL + b * P + shift:SL + b * P + shift + L]
                if kw != padding:
                    patch = patch * mask_rows[kw]
                parts.append(patch)
            rows.append(jnp.concatenate(parts, axis=1) if NB > 1
                        else parts[0])
        return jnp.concatenate(rows, axis=0)

    def conv(w_ref, b_ref):
        acc = None
        for kh in range(k):
            d = jnp.dot(w_ref[:, kh * KH:(kh + 1) * KH],
                        patch_kh(kh),
                        preferred_element_type=jnp.float32)
            acc = d if acc is None else acc + d
        return acc + b_ref[...]

    h1 = conv(wp1_ref, b1_ref)
    h1 = jnp.where(h1 > 0, h1, 0.01 * h1)
    h1 = h1.astype(jnp.bfloat16)
    for b in range(NB):
        base = SL + b * P
        xs_ref[:, base:base + L] = h1[:, b * L:(b + 1) * L]

    out = conv(wp2_ref, b2_ref)
    for b in range(NB):
        out_ref[b] = out[:, b * L:(b + 1) * L]


def kernel(x, w1, b1, w2, b2):
    B, cin, H, W = x.shape
    cout = w2.shape[0]
    k = w1.shape[2]
    padding = (k - 1) // 2
    L = H * W
    K = k * k * cin
    S = padding * W + padding
    SL = ((S + 127) // 128) * 128 if S > 0 else 0
    NB = 2 if B % 2 == 0 else 1

    x_flat = x.reshape(B, cin, L)
    w1f = w1.reshape(cin, K)
    w2f = w2.reshape(cout, K)
    b1v = b1.reshape(cin, 1)
    b2v = b2.reshape(cout, 1)

    _kernel_fn = functools.partial(
        _double_conv_kernel, k=k, padding=padding, W=W, L=L, SL=SL,
        cin=cin, cout=cout, NB=NB)

    flops = 2 * B * K * L * (cin + cout)
    bytes_accessed = 4 * (B * cin * L + B * cout * L + (cin + cout) * (K + 1))

    out_flat = pl.pallas_call(
        _kernel_fn,
        out_shape=jax.ShapeDtypeStruct((B, cout, L), jnp.float32),
        grid=(B // NB,),
        in_specs=[
            pl.BlockSpec((NB, cin, L), lambda b: (b, 0, 0)),
            pl.BlockSpec((cin, K), lambda b: (0, 0)),
            pl.BlockSpec((cin, 1), lambda b: (0, 0)),
            pl.BlockSpec((cout, K), lambda b: (0, 0)),
            pl.BlockSpec((cout, 1), lambda b: (0, 0)),
        ],
        out_specs=pl.BlockSpec((NB, cout, L), lambda b: (b, 0, 0)),
        scratch_shapes=[
            pltpu.VMEM((cin, SL + NB * (L + SL)), jnp.bfloat16),
            pltpu.VMEM((cin, K), jnp.bfloat16),
            pltpu.VMEM((cout, K), jnp.bfloat16),
            pltpu.VMEM((8, L), jnp.bfloat16),
            pltpu.VMEM((8, L), jnp.bfloat16),
        ],
        compiler_params=pltpu.CompilerParams(
            dimension_semantics=("parallel",)),
        cost_estimate=pl.CostEstimate(
            flops=flops, transcendentals=0, bytes_accessed=bytes_accessed),
    )(x_flat, w1f, b1v, w2f, b2v)

    return out_flat.reshape(B, cout, H, W)


# speedup vs baseline: 1.0915x; 1.0056x over previous
import functools

import jax
import jax.numpy as jnp
from jax import lax
from jax.experimental import pallas as pl
from jax.experimental.pallas import tpu as pltpu


def _double_conv_kernel(x_ref, w1_ref, b1_ref, w2_ref, b2_ref, out_ref,
                        xs_ref, wp1_ref, wp2_ref, ml_ref, mr_ref,
                        *, k, padding, W, L, SL, cin, cout, NB, grid2):
    K = k * k * cin
    kk = k * k
    KH = k * cin
    P = SL + L

    if grid2:
        first = pl.program_id(1) == 0
    else:
        pid = pl.program_id(0)
        first = (pid == 0) | (pid == 1) | (pid == pl.num_programs(0) // 2)

    @pl.when(first)
    def _init():
        j = lax.broadcasted_iota(jnp.int32, (K, K), 0)
        m = lax.broadcasted_iota(jnp.int32, (K, K), 1)
        perm = ((j % kk) * cin + (j // kk)) == m
        p = jnp.where(perm, jnp.float32(1), jnp.float32(0)).astype(jnp.bfloat16)
        wp1_ref[...] = jnp.dot(w1_ref[...].astype(jnp.bfloat16), p,
                               preferred_element_type=jnp.float32
                               ).astype(jnp.bfloat16)
        wp2_ref[...] = jnp.dot(w2_ref[...].astype(jnp.bfloat16), p,
                               preferred_element_type=jnp.float32
                               ).astype(jnp.bfloat16)
        lane = lax.broadcasted_iota(jnp.int32, (8, L), 1) % W
        ml_ref[...] = jnp.where(lane >= 1, jnp.float32(1),
                                jnp.float32(0)).astype(jnp.bfloat16)
        mr_ref[...] = jnp.where(lane <= W - 2, jnp.float32(1),
                                jnp.float32(0)).astype(jnp.bfloat16)

    xs_ref[:, 0:SL] = jnp.zeros((cin, SL), jnp.bfloat16)
    for b in range(NB):
        base = SL + b * P
        xs_ref[:, base:base + L] = x_ref[b].astype(jnp.bfloat16)
        xs_ref[:, base + L:base + L + SL] = jnp.zeros((cin, SL), jnp.bfloat16)

    mask_rows = {0: ml_ref[0:1, :], k - 1: mr_ref[0:1, :]}

    def patch_kh(kh):
        rows = []
        for kw in range(k):
            shift = (kh - padding) * W + (kw - padding)
            parts = []
            for b in range(NB):
                patch = xs_ref[:, SL + b * P + shift:SL + b * P + shift + L]
                if kw != padding:
                    patch = patch * mask_rows[kw]
                parts.append(patch)
            rows.append(jnp.concatenate(parts, axis=1) if NB > 1
                        else parts[0])
        return jnp.concatenate(rows, axis=0)

    def conv(w_ref, b_ref):
        acc = None
        for kh in range(k):
            d = jnp.dot(w_ref[:, kh * KH:(kh + 1) * KH],
                        patch_kh(kh),
                        preferred_element_type=jnp.float32)
            acc = d if acc is None else acc + d
        return acc + b_ref[...]

    h1 = conv(wp1_ref, b1_ref)
    h1 = jnp.where(h1 > 0, h1, 0.01 * h1)
    h1 = h1.astype(jnp.bfloat16)
    for b in range(NB):
        base = SL + b * P
        xs_ref[:, base:base + L] = h1[:, b * L:(b + 1) * L]

    out = conv(wp2_ref, b2_ref)
    for b in range(NB):
        out_ref[b] = out[:, b * L:(b + 1) * L]


def kernel(x, w1, b1, w2, b2):
    B, cin, H, W = x.shape
    cout = w2.shape[0]
    k = w1.shape[2]
    padding = (k - 1) // 2
    L = H * W
    K = k * k * cin
    S = padding * W + padding
    SL = ((S + 127) // 128) * 128 if S > 0 else 0
    NB = 2 if B % 2 == 0 else 1
    steps = B // NB
    grid2 = steps % 2 == 0

    x_flat = x.reshape(B, cin, L)
    w1f = w1.reshape(cin, K)
    w2f = w2.reshape(cout, K)
    b1v = b1.reshape(cin, 1)
    b2v = b2.reshape(cout, 1)

    _kernel_fn = functools.partial(
        _double_conv_kernel, k=k, padding=padding, W=W, L=L, SL=SL,
        cin=cin, cout=cout, NB=NB, grid2=grid2)

    flops = 2 * B * K * L * (cin + cout)
    bytes_accessed = 4 * (B * cin * L + B * cout * L + (cin + cout) * (K + 1))

    if grid2:
        GM = steps // 2
        grid = (2, GM)
        im = lambda i, j: (i * GM + j, 0, 0)
        zi = lambda i, j: (0, 0)
        sem = ("parallel", "arbitrary")
    else:
        grid = (steps,)
        im = lambda b: (b, 0, 0)
        zi = lambda b: (0, 0)
        sem = ("parallel",)

    out_flat = pl.pallas_call(
        _kernel_fn,
        out_shape=jax.ShapeDtypeStruct((B, cout, L), jnp.float32),
        grid=grid,
        in_specs=[
            pl.BlockSpec((NB, cin, L), im),
            pl.BlockSpec((cin, K), zi),
            pl.BlockSpec((cin, 1), zi),
            pl.BlockSpec((cout, K), zi),
            pl.BlockSpec((cout, 1), zi),
        ],
        out_specs=pl.BlockSpec((NB, cout, L), im),
        scratch_shapes=[
            pltpu.VMEM((cin, SL + NB * (L + SL)), jnp.bfloat16),
            pltpu.VMEM((cin, K), jnp.bfloat16),
            pltpu.VMEM((cout, K), jnp.bfloat16),
            pltpu.VMEM((8, L), jnp.bfloat16),
            pltpu.VMEM((8, L), jnp.bfloat16),
        ],
        compiler_params=pltpu.CompilerParams(dimension_semantics=sem),
        cost_estimate=pl.CostEstimate(
            flops=flops, transcendentals=0, bytes_accessed=bytes_accessed),
    )(x_flat, w1f, b1v, w2f, b2v)

    return out_flat.reshape(B, cout, H, W)
